# Initial kernel scaffold; baseline (speedup 1.0000x reference)
import jax
import jax.numpy as jnp
from jax.experimental import pallas as pl


def kernel(pred, target):
    raise NotImplementedError("write your pallas kernel here")


# speedup vs baseline: 2.2910x; 2.2910x over previous
import functools

import jax
import jax.numpy as jnp
from jax.experimental import pallas as pl
from jax.experimental.pallas import tpu as pltpu

_SMOOTHING = 0.1


def _body(tgt_ref, pred_ref, out_ref, m_ref, s_ref, sp_ref, pt_ref,
          *, n_classes, n_rows, vb, n_vblocks):
    r = pl.program_id(0)
    k = pl.program_id(1)

    @pl.when(k == 0)
    def _init():
        m_ref[...] = jnp.full_like(m_ref, -jnp.inf)
        s_ref[...] = jnp.zeros_like(s_ref)
        sp_ref[...] = jnp.zeros_like(sp_ref)
        pt_ref[...] = jnp.zeros_like(pt_ref)

    x = pred_ref[...]
    rb = x.shape[0]
    col = k * vb + jax.lax.broadcasted_iota(jnp.int32, x.shape, 1)
    valid = col < n_classes
    xm = jnp.where(valid, x, -jnp.inf)

    bm = jnp.max(xm, axis=1, keepdims=True)
    m_old = m_ref[...]
    m_new = jnp.maximum(m_old, bm)
    s_ref[...] = (s_ref[...] * jnp.exp(m_old - m_new)
                  + jnp.sum(jnp.exp(xm - m_new), axis=1, keepdims=True))
    m_ref[...] = m_new

    sp_ref[...] += jnp.sum(jnp.where(valid, x, 0.0), axis=1, keepdims=True)

    tgt = tgt_ref[0, 0, :].reshape(rb, 1)
    hit = col == tgt
    pt_ref[...] += jnp.sum(jnp.where(hit, x, 0.0), axis=1, keepdims=True)

    @pl.when((r == 0) & (k == 0))
    def _zero_out():
        out_ref[0, 0] = 0.0

    @pl.when(k == n_vblocks - 1)
    def _finalize():
        a = _SMOOTHING / (n_classes - 1)
        b = 1.0 - _SMOOTHING - a
        lse = m_ref[...] + jnp.log(s_ref[...])
        s_row = sp_ref[...] - n_classes * lse
        logp_t = pt_ref[...] - lse
        loss = -(a * s_row + b * logp_t)
        out_ref[0, 0] += jnp.sum(loss) / n_rows


@jax.jit
def kernel(pred, target):
    n_rows, n_classes = pred.shape
    rb = min(n_rows, 256)
    vb = 2048
    n_rblocks = n_rows // rb
    n_vblocks = pl.cdiv(n_classes, vb)

    tgt3 = target.astype(jnp.int32).reshape(n_rblocks, 1, rb)

    out = pl.pallas_call(
        functools.partial(_body, n_classes=n_classes, n_rows=n_rows,
                          vb=vb, n_vblocks=n_vblocks),
        grid=(n_rblocks, n_vblocks),
        in_specs=[
            pl.BlockSpec((1, 1, rb), lambda r, k: (r, 0, 0)),
            pl.BlockSpec((rb, vb), lambda r, k: (r, k)),
        ],
        out_specs=pl.BlockSpec(memory_space=pltpu.SMEM),
        out_shape=jax.ShapeDtypeStruct((1, 1), jnp.float32),
        scratch_shapes=[pltpu.VMEM((rb, 1), jnp.float32) for _ in range(4)],
    )(tgt3, pred)
    return out[0, 0]
